# baseline (speedup 1.0000x reference)
import functools

import jax
import jax.numpy as jnp
from jax.experimental import pallas as pl

_B, _N, _D = 128, 100, 16
_ALPHA = 0.2
_JB = 16
_NL = 128
_F1 = 32
_F2 = 8


def _leaky(v):
    return jnp.maximum(v, _ALPHA * v)


def _mp_kernel(x_ref, feW1_ref, feW2_ref, fnW1_ref, fnW2_ref, bias_ref,
               o_ref, *, bb):
    W1a = feW1_ref[:_D, :]
    W1b = feW1_ref[_D:, :]
    b1 = bias_ref[0, 0:_F1]
    W2T = feW2_ref[...].T
    tiles = jnp.concatenate([W2T] * _JB, axis=1)
    tiles = jnp.concatenate([tiles] * _JB, axis=0)
    rowg = jax.lax.broadcasted_iota(jnp.int32, (_JB * _F2, _JB * _F1), 0)
    colg = jax.lax.broadcasted_iota(jnp.int32, (_JB * _F2, _JB * _F1), 1)
    Wbd_b = jnp.where(rowg // _F2 == colg // _F1, tiles,
                      0.0).astype(jnp.bfloat16)
    b2 = bias_ref[0, _F1:_F1 + _F2]
    b2s = jnp.concatenate([b2] * _JB, axis=0).reshape(_JB * _F2, 1)
    fnW1T = fnW1_ref[...].T
    fnb1 = bias_ref[0, _F1 + _F2:_F1 + _F2 + 32].reshape(32, 1)
    fnW2T = fnW2_ref[...].T
    fnb2 = bias_ref[0, _F1 + _F2 + 32:].reshape(16, 1)

    nfull = _N // _JB
    rem = _N - nfull * _JB

    xTs, PTs, Qs = [], [], []
    for b in range(bb):
        x2d = x_ref[b]
        xz = jnp.concatenate(
            [x2d, jnp.zeros((_NL - _N, _D), jnp.float32)], axis=0)
        x_T = xz.T
        P = jnp.dot(x2d, W1a, preferred_element_type=jnp.float32) + b1
        Q = (jnp.dot(x2d, W1b, preferred_element_type=jnp.float32)
             .astype(jnp.bfloat16))
        Pz = jnp.concatenate(
            [P, jnp.zeros((_NL - _N, _F1), jnp.float32)], axis=0)
        xTs.append(x_T)
        PTs.append(Pz.T.astype(jnp.bfloat16))
        Qs.append(Q)

    accs = [jnp.zeros((_JB * _F2, _NL), jnp.float32) for _ in range(bb)]
    for jb in range(nfull):
        for b in range(bb):
            Qblk = Qs[b][jb * _JB:(jb + 1) * _JB, :]
            Qb3 = jnp.broadcast_to(Qblk[:, :, None], (_JB, _F1, _NL))
            E = _leaky(PTs[b][None, :, :] + Qb3)
            Es = E.reshape(_JB * _F1, _NL)
            Hs = _leaky(jnp.dot(Wbd_b, Es,
                                preferred_element_type=jnp.float32)
                        + b2s)
            accs[b] = accs[b] + Hs
    ATs = []
    for b in range(bb):
        Qblk = Qs[b][nfull * _JB:_N, :]
        Qb3 = jnp.broadcast_to(Qblk[:, :, None], (rem, _F1, _NL))
        E = _leaky(PTs[b][None, :, :] + Qb3)
        Es = E.reshape(rem * _F1, _NL)
        Hr = _leaky(jnp.dot(Wbd_b[:rem * _F2, :rem * _F1], Es,
                            preferred_element_type=jnp.float32)
                    + b2s[:rem * _F2, :])
        ATs.append(accs[b].reshape(_JB, _F2, _NL).sum(axis=0)
                   + Hr.reshape(rem, _F2, _NL).sum(axis=0))

    hin = jnp.concatenate(
        [jnp.concatenate([ATs[b], xTs[b]], axis=0) for b in range(bb)],
        axis=1)
    h = _leaky(jnp.dot(fnW1T, hin,
                       preferred_element_type=jnp.float32) + fnb1)
    o = (jnp.dot(fnW2T, h, preferred_element_type=jnp.float32)
         + fnb2)
    oT = o.T
    for b in range(bb):
        o_ref[b] = oT[b * _NL:b * _NL + _N, :]


def kernel(x, fe_W1, fe_b1, fe_W2, fe_b2, fn_W1, fn_b1, fn_W2, fn_b2):
    bb = 32
    grid = (_B // bb,)
    bias = jnp.concatenate([fe_b1, fe_b2, fn_b1, fn_b2]).reshape(1, -1)
    wspec = lambda r, c: pl.BlockSpec((r, c), lambda i: (0, 0))
    out = pl.pallas_call(
        functools.partial(_mp_kernel, bb=bb),
        grid=grid,
        in_specs=[
            pl.BlockSpec((bb, _N, _D), lambda i: (i, 0, 0)),
            wspec(2 * _D, _F1),
            wspec(_F1, _F2),
            wspec(24, 32),
            wspec(32, 16),
            wspec(1, 88),
        ],
        out_specs=pl.BlockSpec((bb, _N, _D), lambda i: (i, 0, 0)),
        out_shape=jax.ShapeDtypeStruct((_B, _N, _D), jnp.float32),
    )(x, fe_W1, fe_W2, fn_W1, fn_W2, bias)
    return out


# speedup vs baseline: 1.0076x; 1.0076x over previous
import functools

import jax
import jax.numpy as jnp
from jax.experimental import pallas as pl

_B, _N, _D = 128, 100, 16
_ALPHA = 0.2
_JB = 8
_NL = 128
_F1 = 32
_F2 = 8


def _leaky(v):
    return jnp.maximum(v, _ALPHA * v)


def _mp_kernel(x_ref, feW1_ref, feW2_ref, fnW1_ref, fnW2_ref, bias_ref,
               o_ref, *, bb):
    W1a = feW1_ref[:_D, :]
    W1b = feW1_ref[_D:, :]
    b1 = bias_ref[0, 0:_F1]
    W2T = feW2_ref[...].T
    tiles = jnp.concatenate([W2T] * _JB, axis=1)
    tiles = jnp.concatenate([tiles] * _JB, axis=0)
    rowg = jax.lax.broadcasted_iota(jnp.int32, (_JB * _F2, _JB * _F1), 0)
    colg = jax.lax.broadcasted_iota(jnp.int32, (_JB * _F2, _JB * _F1), 1)
    Wbd_b = jnp.where(rowg // _F2 == colg // _F1, tiles,
                      0.0).astype(jnp.bfloat16)
    b2 = bias_ref[0, _F1:_F1 + _F2]
    b2s = jnp.concatenate([b2] * _JB, axis=0).reshape(_JB * _F2, 1)
    fnW1T = fnW1_ref[...].T
    fnb1 = bias_ref[0, _F1 + _F2:_F1 + _F2 + 32].reshape(32, 1)
    fnW2T = fnW2_ref[...].T
    fnb2 = bias_ref[0, _F1 + _F2 + 32:].reshape(16, 1)

    nfull = _N // _JB
    rem = _N - nfull * _JB

    xTs, PTs, Qs = [], [], []
    for b in range(bb):
        x2d = x_ref[b]
        xz = jnp.concatenate(
            [x2d, jnp.zeros((_NL - _N, _D), jnp.float32)], axis=0)
        x_T = xz.T
        P = jnp.dot(x2d, W1a, preferred_element_type=jnp.float32) + b1
        Q = (jnp.dot(x2d, W1b, preferred_element_type=jnp.float32)
             .astype(jnp.bfloat16))
        Pz = jnp.concatenate(
            [P, jnp.zeros((_NL - _N, _F1), jnp.float32)], axis=0)
        xTs.append(x_T)
        P_Tb = Pz.T.astype(jnp.bfloat16)
        PTs.append(jnp.concatenate([P_Tb] * _JB, axis=0))
        Qs.append(Q)

    accs = [jnp.zeros((_JB * _F2, _NL), jnp.float32) for _ in range(bb)]
    for jb in range(nfull):
        for b in range(bb):
            Qblk = Qs[b][jb * _JB:(jb + 1) * _JB, :]
            Qb3 = jnp.broadcast_to(Qblk[:, :, None], (_JB, _F1, _NL))
            Es = _leaky(PTs[b] + Qb3.reshape(_JB * _F1, _NL))
            Hs = _leaky(jnp.dot(Wbd_b, Es,
                                preferred_element_type=jnp.float32)
                        + b2s)
            accs[b] = accs[b] + Hs
    ATs = []
    for b in range(bb):
        Qblk = Qs[b][nfull * _JB:_N, :]
        Qb3 = jnp.broadcast_to(Qblk[:, :, None], (rem, _F1, _NL))
        Es = _leaky(PTs[b][:rem * _F1, :] + Qb3.reshape(rem * _F1, _NL))
        Hr = _leaky(jnp.dot(Wbd_b[:rem * _F2, :rem * _F1], Es,
                            preferred_element_type=jnp.float32)
                    + b2s[:rem * _F2, :])
        ATs.append(accs[b].reshape(_JB, _F2, _NL).sum(axis=0)
                   + Hr.reshape(rem, _F2, _NL).sum(axis=0))

    hin = jnp.concatenate(
        [jnp.concatenate([ATs[b], xTs[b]], axis=0) for b in range(bb)],
        axis=1)
    h = _leaky(jnp.dot(fnW1T, hin,
                       preferred_element_type=jnp.float32) + fnb1)
    o = (jnp.dot(fnW2T, h, preferred_element_type=jnp.float32)
         + fnb2)
    oT = o.T
    for b in range(bb):
        o_ref[b] = oT[b * _NL:b * _NL + _N, :]


def kernel(x, fe_W1, fe_b1, fe_W2, fe_b2, fn_W1, fn_b1, fn_W2, fn_b2):
    bb = 32
    grid = (_B // bb,)
    bias = jnp.concatenate([fe_b1, fe_b2, fn_b1, fn_b2]).reshape(1, -1)
    wspec = lambda r, c: pl.BlockSpec((r, c), lambda i: (0, 0))
    out = pl.pallas_call(
        functools.partial(_mp_kernel, bb=bb),
        grid=grid,
        in_specs=[
            pl.BlockSpec((bb, _N, _D), lambda i: (i, 0, 0)),
            wspec(2 * _D, _F1),
            wspec(_F1, _F2),
            wspec(24, 32),
            wspec(32, 16),
            wspec(1, 88),
        ],
        out_specs=pl.BlockSpec((bb, _N, _D), lambda i: (i, 0, 0)),
        out_shape=jax.ShapeDtypeStruct((_B, _N, _D), jnp.float32),
    )(x, fe_W1, fe_W2, fn_W1, fn_W2, bias)
    return out


# speedup vs baseline: 1.0342x; 1.0264x over previous
import functools

import jax
import jax.numpy as jnp
from jax.experimental import pallas as pl

_B, _N, _D = 128, 100, 16
_ALPHA = 0.2
_JB = 8
_NL = 128
_F1 = 32
_F2 = 8


def _leaky(v):
    return jnp.maximum(v, _ALPHA * v)


def _mp_kernel(x_ref, feW1_ref, feW2_ref, fnW1_ref, fnW2_ref, feb1_ref,
               feb2_ref, fnb1_ref, fnb2_ref, o_ref, *, bb):
    W1a = feW1_ref[:_D, :]
    W1b = feW1_ref[_D:, :]
    b1 = feb1_ref[...]
    W2T = feW2_ref[...].T
    tiles = jnp.concatenate([W2T] * _JB, axis=1)
    tiles = jnp.concatenate([tiles] * _JB, axis=0)
    rowg = jax.lax.broadcasted_iota(jnp.int32, (_JB * _F2, _JB * _F1), 0)
    colg = jax.lax.broadcasted_iota(jnp.int32, (_JB * _F2, _JB * _F1), 1)
    Wbd_b = jnp.where(rowg // _F2 == colg // _F1, tiles,
                      0.0).astype(jnp.bfloat16)
    b2 = feb2_ref[...]
    b2s = jnp.concatenate([b2] * _JB, axis=0).reshape(_JB * _F2, 1)
    fnW1T = fnW1_ref[...].T
    fnb1 = fnb1_ref[...].reshape(32, 1)
    fnW2T = fnW2_ref[...].T
    fnb2 = fnb2_ref[...].reshape(16, 1)

    nfull = _N // _JB
    rem = _N - nfull * _JB

    xTs, PTs, Qs = [], [], []
    for b in range(bb):
        x2d = x_ref[b]
        xz = jnp.concatenate(
            [x2d, jnp.zeros((_NL - _N, _D), jnp.float32)], axis=0)
        x_T = xz.T
        P = jnp.dot(x2d, W1a, preferred_element_type=jnp.float32) + b1
        Q = (jnp.dot(x2d, W1b, preferred_element_type=jnp.float32)
             .astype(jnp.bfloat16))
        Pz = jnp.concatenate(
            [P, jnp.zeros((_NL - _N, _F1), jnp.float32)], axis=0)
        xTs.append(x_T)
        P_Tb = Pz.T.astype(jnp.bfloat16)
        PTs.append(jnp.concatenate([P_Tb] * _JB, axis=0))
        Qs.append(Q)

    accs = [jnp.zeros((_JB * _F2, _NL), jnp.float32) for _ in range(bb)]
    for jb in range(nfull):
        for b in range(bb):
            Qblk = Qs[b][jb * _JB:(jb + 1) * _JB, :]
            Qb3 = jnp.broadcast_to(Qblk[:, :, None], (_JB, _F1, _NL))
            Es = _leaky(PTs[b] + Qb3.reshape(_JB * _F1, _NL))
            Hs = _leaky(jnp.dot(Wbd_b, Es,
                                preferred_element_type=jnp.float32)
                        + b2s)
            accs[b] = accs[b] + Hs
    ATs = []
    for b in range(bb):
        Qblk = Qs[b][nfull * _JB:_N, :]
        Qb3 = jnp.broadcast_to(Qblk[:, :, None], (rem, _F1, _NL))
        Es = _leaky(PTs[b][:rem * _F1, :] + Qb3.reshape(rem * _F1, _NL))
        Hr = _leaky(jnp.dot(Wbd_b[:rem * _F2, :rem * _F1], Es,
                            preferred_element_type=jnp.float32)
                    + b2s[:rem * _F2, :])
        ATs.append(accs[b].reshape(_JB, _F2, _NL).sum(axis=0)
                   + Hr.reshape(rem, _F2, _NL).sum(axis=0))

    hin = jnp.concatenate(
        [jnp.concatenate([ATs[b], xTs[b]], axis=0) for b in range(bb)],
        axis=1)
    h = _leaky(jnp.dot(fnW1T, hin,
                       preferred_element_type=jnp.float32) + fnb1)
    o = (jnp.dot(fnW2T, h, preferred_element_type=jnp.float32)
         + fnb2)
    oT = o.T
    for b in range(bb):
        o_ref[b] = oT[b * _NL:b * _NL + _N, :]


def kernel(x, fe_W1, fe_b1, fe_W2, fe_b2, fn_W1, fn_b1, fn_W2, fn_b2):
    bb = 32
    grid = (_B // bb,)
    wspec = lambda r, c: pl.BlockSpec((r, c), lambda i: (0, 0))
    out = pl.pallas_call(
        functools.partial(_mp_kernel, bb=bb),
        grid=grid,
        in_specs=[
            pl.BlockSpec((bb, _N, _D), lambda i: (i, 0, 0)),
            wspec(2 * _D, _F1),
            wspec(_F1, _F2),
            wspec(24, 32),
            wspec(32, 16),
            pl.BlockSpec((_F1,), lambda i: (0,)),
            pl.BlockSpec((_F2,), lambda i: (0,)),
            pl.BlockSpec((32,), lambda i: (0,)),
            pl.BlockSpec((16,), lambda i: (0,)),
        ],
        out_specs=pl.BlockSpec((bb, _N, _D), lambda i: (i, 0, 0)),
        out_shape=jax.ShapeDtypeStruct((_B, _N, _D), jnp.float32),
    )(x, fe_W1, fe_W2, fn_W1, fn_W2, fe_b1, fe_b2, fn_b1, fn_b2)
    return out
